# baseline (speedup 1.0000x reference)
import functools

import jax
import jax.numpy as jnp
from jax import lax
from jax.experimental import pallas as pl
from jax.experimental.pallas import tpu as pltpu
from jax.experimental.pallas import tpu_sc as plsc

_N = 10000
_E = 320000
_D = 128
_NC = 2
_NS = 16
_NW = _NC * _NS
_EPT = _E // _NW
_C = 50
_NCHUNK = _EPT // _C
_NPAIR = _NCHUNK // 2
_SEG = 56
_REC = 4 * _SEG + 16
_ZC = 40
_CPT = 10
_RPT = _N // _CPT
_ZR = 200

_INV_SQRT2 = 0.7071067811865476


def _gelu(u):
    z = jnp.abs(u) * _INV_SQRT2
    t = 1.0 / (1.0 + 0.3275911 * z)
    poly = t * (0.254829592 + t * (-0.284496736 + t * (1.421413741
                + t * (-1.453152027 + t * 1.061405429))))
    erf = jnp.sign(u) * (1.0 - poly * jnp.exp(-z * z))
    return 0.5 * u * (1.0 + erf)


_TG_A = 1.5957691216057308
_TG_B = _TG_A * 0.044715


def _gelu_fast(u):
    y = u * (_TG_A + _TG_B * (u * u))
    return u / (1.0 + jnp.exp(-y))


_sc_mesh = plsc.VectorSubcoreMesh(core_axis_name="c", subcore_axis_name="s")


@functools.partial(
    pl.kernel,
    mesh=_sc_mesh,
    out_type=jax.ShapeDtypeStruct((_NC, _N, _D), jnp.float32),
    scratch_types=[
        pltpu.VMEM((_REC,), jnp.int32),
        pltpu.VMEM((_REC,), jnp.int32),
        pltpu.VMEM((2, _C), jnp.int32),
        pltpu.VMEM((2, _C), jnp.int32),
        pltpu.VMEM((_C, _D), jnp.float32),
        pltpu.VMEM((_C, _D), jnp.float32),
        pltpu.VMEM((60, _D), jnp.float32),
        pltpu.VMEM((_ZC, _D), jnp.float32),
        pltpu.VMEM_SHARED((_N, _D), jnp.float32),
        pltpu.SemaphoreType.DMA,
        pltpu.SemaphoreType.DMA,
        pltpu.SemaphoreType.DMA,
        pltpu.SemaphoreType.DMA,
        pltpu.SemaphoreType.DMA,
        pltpu.SemaphoreType.DMA,
    ],
)
def _edge_kernel(h_hbm, combo_hbm, sc_hbm, dst_hbm, out_hbm,
                 ipa, ipb, dpa, dpb, xa, xb, combo_v, zbuf, agg_sh,
                 sxa, sxb, sia, sib, sda, sdb):
    c = lax.axis_index("c")
    s = lax.axis_index("s")
    wid = c * _NS + s

    pltpu.sync_copy(combo_hbm, combo_v)

    zero16 = jnp.zeros((16,), jnp.float32)

    def zrow(r, carry):
        for j in range(_D // 16):
            zbuf[r, pl.ds(j * 16, 16)] = zero16
        return carry

    lax.fori_loop(0, _ZC, zrow, 0)

    @pl.when(s < _CPT)
    def _zero_agg():
        for t in range(_RPT // _ZC):
            pltpu.sync_copy(zbuf,
                            agg_sh.at[pl.ds(s * _RPT + t * _ZC, _ZC)])

    plsc.subcore_barrier()

    def gather(ip, j, xdst, semx):
        pltpu.async_copy(h_hbm.at[ip.at[pl.ds(j * _SEG, _C)]], xdst, semx)

    def wait_gather(ip, j, xdst, semx):
        pltpu.make_async_copy(h_hbm.at[ip.at[pl.ds(j * _SEG, _C)]],
                              xdst, semx).wait()

    def compute_scatter(ip, dp, j, xbuf):
        cbase = (2 + j) * _SEG

        def row2(i, rc):
            for rr in range(2):
                r = i * 2 + rr
                code = ip[pl.ds(cbase + r, 16)][0]
                for jj in range(_D // 16):
                    sl = pl.ds(jj * 16, 16)
                    xbuf[r, sl] = _gelu_fast(xbuf[r, sl] + combo_v[code, sl])
            return rc

        lax.fori_loop(0, _C // 2, row2, 0)
        pltpu.sync_copy(xbuf, agg_sh.at[dp.at[j]], add=True)

    pltpu.sync_copy(sc_hbm.at[wid, 0], ipa)
    pltpu.sync_copy(dst_hbm.at[wid, 0], dpa)
    gather(ipa, 0, xa, sxa)

    def body(p, cur, dcur, nxt, dnxt, sem_nxt, sem_dnxt):
        @pl.when(p + 1 < _NPAIR)
        def _stage_next():
            pltpu.async_copy(sc_hbm.at[wid, p + 1], nxt, sem_nxt)
            pltpu.async_copy(dst_hbm.at[wid, p + 1], dnxt, sem_dnxt)

        gather(cur, 1, xb, sxb)
        wait_gather(cur, 0, xa, sxa)
        compute_scatter(cur, dcur, 0, xa)

        @pl.when(p + 1 < _NPAIR)
        def _next_gather():
            pltpu.make_async_copy(sc_hbm.at[wid, p + 1], nxt, sem_nxt).wait()
            pltpu.make_async_copy(dst_hbm.at[wid, p + 1], dnxt,
                                  sem_dnxt).wait()
            gather(nxt, 0, xa, sxa)

        wait_gather(cur, 1, xb, sxb)
        compute_scatter(cur, dcur, 1, xb)

    def pairpair(i, carry):
        body(2 * i, ipa, dpa, ipb, dpb, sib, sdb)
        body(2 * i + 1, ipb, dpb, ipa, dpa, sia, sda)
        return carry

    lax.fori_loop(0, _NPAIR // 2, pairpair, 0)

    plsc.subcore_barrier()

    @pl.when(s < _CPT)
    def _copy_out():
        pltpu.sync_copy(agg_sh.at[pl.ds(s * _RPT, _RPT)],
                        out_hbm.at[c, pl.ds(s * _RPT, _RPT)])


_NB = 10
_BR = _N // _NB


def _node_body(x_ref, agg_ref, w1_ref, b1_ref, w2_ref, b2_ref,
               gam_ref, bet_ref, eps_ref, out_ref, h2_scr, ssum, ssq):
    p = pl.program_id(0)
    b = pl.program_id(1)

    @pl.when(p == 0)
    def _pass0():
        hb = ((1.0 + eps_ref[0, 0]) * x_ref[...]
              + agg_ref[0] + agg_ref[1])
        a1 = _gelu(jnp.dot(hb, w1_ref[...],
                           preferred_element_type=jnp.float32) + b1_ref[...])
        h2 = jnp.dot(a1, w2_ref[...],
                     preferred_element_type=jnp.float32) + b2_ref[...]
        h2_scr[pl.ds(b * _BR, _BR), :] = h2
        colsum = jnp.sum(h2, axis=0, keepdims=True)
        colsq = jnp.sum(h2 * h2, axis=0, keepdims=True)

        @pl.when(b == 0)
        def _init():
            ssum[...] = colsum
            ssq[...] = colsq

        @pl.when(b != 0)
        def _acc():
            ssum[...] += colsum
            ssq[...] += colsq

    @pl.when(p == 1)
    def _pass1():
        mu = ssum[...] / _N
        var = ssq[...] / _N - mu * mu
        h2 = h2_scr[pl.ds(b * _BR, _BR), :]
        g = (h2 - mu) * gam_ref[...] * lax.rsqrt(var + 1e-5) + bet_ref[...]
        out_ref[...] = (x_ref[...] + _gelu(g)) * _INV_SQRT2


_node_call = pl.pallas_call(
    _node_body,
    grid=(2, _NB),
    in_specs=[
        pl.BlockSpec((_BR, _D), lambda p, b: (b, 0)),
        pl.BlockSpec((_NC, _BR, _D), lambda p, b: (0, b, 0)),
        pl.BlockSpec((_D, _D), lambda p, b: (0, 0)),
        pl.BlockSpec((1, _D), lambda p, b: (0, 0)),
        pl.BlockSpec((_D, _D), lambda p, b: (0, 0)),
        pl.BlockSpec((1, _D), lambda p, b: (0, 0)),
        pl.BlockSpec((1, _D), lambda p, b: (0, 0)),
        pl.BlockSpec((1, _D), lambda p, b: (0, 0)),
        pl.BlockSpec((1, 1), lambda p, b: (0, 0)),
    ],
    out_specs=pl.BlockSpec((_BR, _D), lambda p, b: (b, 0)),
    out_shape=jax.ShapeDtypeStruct((_N, _D), jnp.float32),
    scratch_shapes=[
        pltpu.VMEM((_N, _D), jnp.float32),
        pltpu.VMEM((1, _D), jnp.float32),
        pltpu.VMEM((1, _D), jnp.float32),
    ],
)


def kernel(x, edge_index, edge_attr, params):
    code = edge_attr[:, 0] * 12 + edge_attr[:, 1] * 2 + edge_attr[:, 2]
    srcr = edge_index[0].reshape(_NW, _NPAIR, 2, _C)
    coder = code.reshape(_NW, _NPAIR, 2, _C)
    segs = jnp.concatenate([srcr, coder], axis=2)
    segs = jnp.pad(segs, ((0, 0), (0, 0), (0, 0), (0, _SEG - _C)))
    sc_packed = jnp.pad(segs.reshape(_NW, _NPAIR, 4 * _SEG),
                        ((0, 0), (0, 0), (0, 16)))
    dst3 = edge_index[1].reshape(_NW, _NPAIR, 2, _C)
    h = x
    for p in params:
        combo = (p['tab0'][:, None, None, :]
                 + p['tab1'][None, :, None, :]
                 + p['tab2'][None, None, :, :]).reshape(60, _D)
        agg2 = _edge_kernel(h, combo, sc_packed, dst3)
        h = _node_call(h, agg2,
                       p['W1'], p['b1'].reshape(1, _D),
                       p['W2'], p['b2'].reshape(1, _D),
                       p['gamma'].reshape(1, _D), p['beta'].reshape(1, _D),
                       p['eps'].reshape(1, 1))
    return h


# speedup vs baseline: 6.1518x; 6.1518x over previous
import functools

import jax
import jax.numpy as jnp
from jax import lax
from jax.experimental import pallas as pl
from jax.experimental.pallas import tpu as pltpu
from jax.experimental.pallas import tpu_sc as plsc

_N = 10000
_E = 320000
_D = 128
_NC = 2
_NS = 16
_NW = _NC * _NS
_EPT = _E // _NW
_C = 50
_NCHUNK = _EPT // _C
_NPAIR = _NCHUNK // 2
_SEG = 56
_REC = 4 * _SEG + 16
_ZC = 40
_CPT = 10
_RPT = _N // _CPT
_ZR = 200

_INV_SQRT2 = 0.7071067811865476


def _gelu(u):
    z = jnp.abs(u) * _INV_SQRT2
    t = 1.0 / (1.0 + 0.3275911 * z)
    poly = t * (0.254829592 + t * (-0.284496736 + t * (1.421413741
                + t * (-1.453152027 + t * 1.061405429))))
    erf = jnp.sign(u) * (1.0 - poly * jnp.exp(-z * z))
    return 0.5 * u * (1.0 + erf)


_TG_A = 1.5957691216057308
_TG_B = _TG_A * 0.044715


def _gelu_fast(u):
    y = u * (_TG_A + _TG_B * (u * u))
    return u / (1.0 + jnp.exp(-y))


_sc_mesh = plsc.VectorSubcoreMesh(core_axis_name="c", subcore_axis_name="s")


@functools.partial(
    pl.kernel,
    mesh=_sc_mesh,
    out_type=jax.ShapeDtypeStruct((_NC, _N, _D), jnp.float32),
    scratch_types=[
        pltpu.VMEM((_REC,), jnp.int32),
        pltpu.VMEM((_REC,), jnp.int32),
        pltpu.VMEM((2, _C), jnp.int32),
        pltpu.VMEM((2, _C), jnp.int32),
        pltpu.VMEM((_C, _D), jnp.float32),
        pltpu.VMEM((_C, _D), jnp.float32),
        pltpu.VMEM((_C, _D), jnp.float32),
        pltpu.VMEM((_C, _D), jnp.float32),
        pltpu.VMEM_SHARED((60, _D), jnp.float32),
        pltpu.VMEM((_ZC, _D), jnp.float32),
        pltpu.VMEM_SHARED((_N, _D), jnp.float32),
        pltpu.SemaphoreType.DMA,
        pltpu.SemaphoreType.DMA,
        pltpu.SemaphoreType.DMA,
        pltpu.SemaphoreType.DMA,
        pltpu.SemaphoreType.DMA,
        pltpu.SemaphoreType.DMA,
        pltpu.SemaphoreType.DMA,
        pltpu.SemaphoreType.DMA,
    ],
)
def _edge_kernel(h_hbm, combo_hbm, sc_hbm, dst_hbm, out_hbm,
                 ipa, ipb, dpa, dpb, xa, xb, ca, cb, combo_v, zbuf, agg_sh,
                 sxa, sxb, sca, scb, sia, sib, sda, sdb):
    c = lax.axis_index("c")
    s = lax.axis_index("s")
    wid = c * _NS + s

    @pl.when(s == 0)
    def _stage_combo():
        pltpu.sync_copy(combo_hbm, combo_v)

    zero16 = jnp.zeros((16,), jnp.float32)

    def zrow(r, carry):
        for j in range(_D // 16):
            zbuf[r, pl.ds(j * 16, 16)] = zero16
        return carry

    lax.fori_loop(0, _ZC, zrow, 0)

    @pl.when(s < _CPT)
    def _zero_agg():
        for t in range(_RPT // _ZC):
            pltpu.sync_copy(zbuf,
                            agg_sh.at[pl.ds(s * _RPT + t * _ZC, _ZC)])

    plsc.subcore_barrier()

    def gather(ip, j, xdst, cdst, semx, semc):
        pltpu.async_copy(h_hbm.at[ip.at[pl.ds(j * _SEG, _C)]], xdst, semx)
        pltpu.async_copy(combo_v.at[ip.at[pl.ds((2 + j) * _SEG, _C)]],
                         cdst, semc)

    def wait_gather(ip, j, xdst, cdst, semx, semc):
        pltpu.make_async_copy(h_hbm.at[ip.at[pl.ds(j * _SEG, _C)]],
                              xdst, semx).wait()
        pltpu.make_async_copy(combo_v.at[ip.at[pl.ds((2 + j) * _SEG, _C)]],
                              cdst, semc).wait()

    def compute_scatter(dp, j, xbuf, cbuf):
        def row2(i, rc):
            for rr in range(2):
                r = i * 2 + rr
                for jj in range(_D // 16):
                    sl = pl.ds(jj * 16, 16)
                    xbuf[r, sl] = _gelu_fast(xbuf[r, sl] + cbuf[r, sl])
            return rc

        lax.fori_loop(0, _C // 2, row2, 0)
        pltpu.sync_copy(xbuf, agg_sh.at[dp.at[j]], add=True)

    pltpu.sync_copy(sc_hbm.at[wid, 0], ipa)
    pltpu.sync_copy(dst_hbm.at[wid, 0], dpa)
    gather(ipa, 0, xa, ca, sxa, sca)

    def body(p, cur, dcur, nxt, dnxt, sem_nxt, sem_dnxt):
        @pl.when(p + 1 < _NPAIR)
        def _stage_next():
            pltpu.async_copy(sc_hbm.at[wid, p + 1], nxt, sem_nxt)
            pltpu.async_copy(dst_hbm.at[wid, p + 1], dnxt, sem_dnxt)

        gather(cur, 1, xb, cb, sxb, scb)
        wait_gather(cur, 0, xa, ca, sxa, sca)
        compute_scatter(dcur, 0, xa, ca)

        @pl.when(p + 1 < _NPAIR)
        def _next_gather():
            pltpu.make_async_copy(sc_hbm.at[wid, p + 1], nxt, sem_nxt).wait()
            pltpu.make_async_copy(dst_hbm.at[wid, p + 1], dnxt,
                                  sem_dnxt).wait()
            gather(nxt, 0, xa, ca, sxa, sca)

        wait_gather(cur, 1, xb, cb, sxb, scb)
        compute_scatter(dcur, 1, xb, cb)

    def pairpair(i, carry):
        body(2 * i, ipa, dpa, ipb, dpb, sib, sdb)
        body(2 * i + 1, ipb, dpb, ipa, dpa, sia, sda)
        return carry

    lax.fori_loop(0, _NPAIR // 2, pairpair, 0)

    plsc.subcore_barrier()

    @pl.when(s < _CPT)
    def _copy_out():
        pltpu.sync_copy(agg_sh.at[pl.ds(s * _RPT, _RPT)],
                        out_hbm.at[c, pl.ds(s * _RPT, _RPT)])


_NB = 10
_BR = _N // _NB


def _node_body(x_ref, agg_ref, w1_ref, b1_ref, w2_ref, b2_ref,
               gam_ref, bet_ref, eps_ref, out_ref, h2_scr, ssum, ssq):
    p = pl.program_id(0)
    b = pl.program_id(1)

    @pl.when(p == 0)
    def _pass0():
        hb = ((1.0 + eps_ref[0, 0]) * x_ref[...]
              + agg_ref[0] + agg_ref[1])
        a1 = _gelu(jnp.dot(hb.astype(jnp.bfloat16), w1_ref[...],
                           preferred_element_type=jnp.float32) + b1_ref[...])
        h2 = jnp.dot(a1.astype(jnp.bfloat16), w2_ref[...],
                     preferred_element_type=jnp.float32) + b2_ref[...]
        h2_scr[pl.ds(b * _BR, _BR), :] = h2
        colsum = jnp.sum(h2, axis=0, keepdims=True)
        colsq = jnp.sum(h2 * h2, axis=0, keepdims=True)

        @pl.when(b == 0)
        def _init():
            ssum[...] = colsum
            ssq[...] = colsq

        @pl.when(b != 0)
        def _acc():
            ssum[...] += colsum
            ssq[...] += colsq

    @pl.when(p == 1)
    def _pass1():
        mu = ssum[...] / _N
        var = ssq[...] / _N - mu * mu
        h2 = h2_scr[pl.ds(b * _BR, _BR), :]
        g = (h2 - mu) * gam_ref[...] * lax.rsqrt(var + 1e-5) + bet_ref[...]
        out_ref[...] = (x_ref[...] + _gelu(g)) * _INV_SQRT2


_node_call = pl.pallas_call(
    _node_body,
    grid=(2, _NB),
    in_specs=[
        pl.BlockSpec((_BR, _D), lambda p, b: (b, 0)),
        pl.BlockSpec((_NC, _BR, _D), lambda p, b: (0, b, 0)),
        pl.BlockSpec((_D, _D), lambda p, b: (0, 0)),
        pl.BlockSpec((1, _D), lambda p, b: (0, 0)),
        pl.BlockSpec((_D, _D), lambda p, b: (0, 0)),
        pl.BlockSpec((1, _D), lambda p, b: (0, 0)),
        pl.BlockSpec((1, _D), lambda p, b: (0, 0)),
        pl.BlockSpec((1, _D), lambda p, b: (0, 0)),
        pl.BlockSpec((1, 1), lambda p, b: (0, 0)),
    ],
    out_specs=pl.BlockSpec((_BR, _D), lambda p, b: (b, 0)),
    out_shape=jax.ShapeDtypeStruct((_N, _D), jnp.float32),
    scratch_shapes=[
        pltpu.VMEM((_N, _D), jnp.float32),
        pltpu.VMEM((1, _D), jnp.float32),
        pltpu.VMEM((1, _D), jnp.float32),
    ],
)


def kernel(x, edge_index, edge_attr, params):
    code = edge_attr[:, 0] * 12 + edge_attr[:, 1] * 2 + edge_attr[:, 2]
    srcr = edge_index[0].reshape(_NW, _NPAIR, 2, _C)
    coder = code.reshape(_NW, _NPAIR, 2, _C)
    segs = jnp.concatenate([srcr, coder], axis=2)
    segs = jnp.pad(segs, ((0, 0), (0, 0), (0, 0), (0, _SEG - _C)))
    sc_packed = jnp.pad(segs.reshape(_NW, _NPAIR, 4 * _SEG),
                        ((0, 0), (0, 0), (0, 16)))
    dst3 = edge_index[1].reshape(_NW, _NPAIR, 2, _C)
    h = x
    for p in params:
        combo = (p['tab0'][:, None, None, :]
                 + p['tab1'][None, :, None, :]
                 + p['tab2'][None, None, :, :]).reshape(60, _D)
        agg2 = _edge_kernel(h, combo, sc_packed, dst3)
        h = _node_call(h, agg2,
                       p['W1'].astype(jnp.bfloat16), p['b1'].reshape(1, _D),
                       p['W2'].astype(jnp.bfloat16), p['b2'].reshape(1, _D),
                       p['gamma'].reshape(1, _D), p['beta'].reshape(1, _D),
                       p['eps'].reshape(1, 1))
    return h
